# baseline (speedup 1.0000x reference)
import jax
import jax.numpy as jnp
from jax.experimental import pallas as pl
from jax.experimental.pallas import tpu as pltpu

_D = 256


def _body(row_ref, col_ref, out_ref, plane_ref, sem):
    h = 32
    w = 32
    hw = h * w
    b = out_ref.shape[0]
    lane = jax.lax.broadcasted_iota(jnp.int32, (w, hw), 1)
    idx0 = jax.lax.broadcasted_iota(jnp.int32, (w, hw), 0)
    s_col = jnp.where((lane & (w - 1)) == idx0, 1.0, 0.0).astype(jnp.float32)
    s_row = jnp.where((lane >> 5) == idx0, 1.0, 0.0).astype(jnp.float32)
    dims = (((0,), (0,)), ((), ()))
    plane_ref[:_D] = jax.lax.dot_general(
        col_ref[:w, :], s_col, dims,
        precision=jax.lax.Precision.HIGHEST,
        preferred_element_type=jnp.float32,
    )
    plane_ref[_D:] = jax.lax.dot_general(
        row_ref[:h, :], s_row, dims,
        precision=jax.lax.Precision.HIGHEST,
        preferred_element_type=jnp.float32,
    )
    for i in range(b):
        pltpu.make_async_copy(plane_ref, out_ref.at[i], sem).start()
    pltpu.make_async_copy(out_ref, out_ref, sem).wait()


def kernel(x, row_embed, col_embed):
    b = x.shape[0]
    h, w = x.shape[-2], x.shape[-1]
    out = pl.pallas_call(
        _body,
        in_specs=[
            pl.BlockSpec(memory_space=pltpu.VMEM),
            pl.BlockSpec(memory_space=pltpu.VMEM),
        ],
        out_specs=pl.BlockSpec(memory_space=pl.ANY),
        out_shape=jax.ShapeDtypeStruct((b, 2 * _D, h * w), jnp.float32),
        scratch_shapes=[
            pltpu.VMEM((2 * _D, h * w), jnp.float32),
            pltpu.SemaphoreType.DMA,
        ],
    )(row_embed, col_embed)
    return out.reshape(b, 2 * _D, h, w)


# speedup vs baseline: 1.0039x; 1.0039x over previous
import jax
import jax.numpy as jnp
from jax.experimental import pallas as pl
from jax.experimental.pallas import tpu as pltpu

_D = 256


def _body(row_ref, col_ref, out_ref, plane_ref, sem):
    h = 32
    w = 32
    hw = h * w
    b = out_ref.shape[0]
    lane = jax.lax.broadcasted_iota(jnp.int32, (w, hw), 1)
    idx0 = jax.lax.broadcasted_iota(jnp.int32, (w, hw), 0)
    s_col = jnp.where((lane & (w - 1)) == idx0, 1.0, 0.0).astype(jnp.float32)
    s_row = jnp.where((lane >> 5) == idx0, 1.0, 0.0).astype(jnp.float32)
    dims = (((0,), (0,)), ((), ()))
    plane_ref[:_D] = jax.lax.dot_general(
        col_ref[:w, :], s_col, dims,
        precision=jax.lax.Precision.HIGHEST,
        preferred_element_type=jnp.float32,
    )
    plane_ref[_D:] = jax.lax.dot_general(
        row_ref[:h, :], s_row, dims,
        precision=jax.lax.Precision.HIGHEST,
        preferred_element_type=jnp.float32,
    )
    half = _D
    for i in range(b):
        pltpu.make_async_copy(
            plane_ref.at[pl.ds(0, half)], out_ref.at[i, pl.ds(0, half)], sem
        ).start()
        pltpu.make_async_copy(
            plane_ref.at[pl.ds(half, half)], out_ref.at[i, pl.ds(half, half)], sem
        ).start()
    pltpu.make_async_copy(out_ref, out_ref, sem).wait()


def kernel(x, row_embed, col_embed):
    b = x.shape[0]
    h, w = x.shape[-2], x.shape[-1]
    out = pl.pallas_call(
        _body,
        in_specs=[
            pl.BlockSpec(memory_space=pltpu.VMEM),
            pl.BlockSpec(memory_space=pltpu.VMEM),
        ],
        out_specs=pl.BlockSpec(memory_space=pl.ANY),
        out_shape=jax.ShapeDtypeStruct((b, 2 * _D, h * w), jnp.float32),
        scratch_shapes=[
            pltpu.VMEM((2 * _D, h * w), jnp.float32),
            pltpu.SemaphoreType.DMA,
        ],
    )(row_embed, col_embed)
    return out.reshape(b, 2 * _D, h, w)
